# baseline (speedup 1.0000x reference)
import jax
import jax.numpy as jnp
from jax import lax
from jax.experimental import pallas as pl
from jax.experimental.pallas import tpu as pltpu
from jax.experimental.pallas import tpu_sc as plsc

D = 64
B = 16384
NW = 32
RPW = B // NW
GRP = RPW // 16
ROWW = 24 * D
TROW = 128

CLS0 = 0
WPAY = 64
WNUM = 128
BLN = 192
ANUM = 256
C0NSPL = ANUM + 896
C1NSPL = C0NSPL + 224
C2PSPL = C1NSPL + 224
C2NSPL = C2PSPL + 16
NCONST = C2NSPL + 16
OFFC = (1, 3, 10)
PAY0 = 14


def _rsqrt16(x):
    i = plsc.bitcast(x, jnp.int32)
    i = jnp.int32(0x5F3759DF) - (i >> 1)
    y = plsc.bitcast(i, jnp.float32)
    return y * (1.5 - (x * 0.5) * y * y)


def _sc_body(tab_hbm, srep_hbm, ic_hbm, ip_hbm, c_hbm, out_hbm,
             icv, ipv, cv, ob, gbuf, sbuf, idsall, sems, sem_in, sem_s):
    wid = lax.axis_index("s") * 2 + lax.axis_index("c")
    base = wid * RPW
    pltpu.sync_copy(c_hbm, cv)
    pltpu.sync_copy(ic_hbm.at[:, pl.ds(base, RPW)], icv)
    pltpu.sync_copy(ip_hbm.at[:, pl.ds(base, RPW)], ipv)

    for row in range(32):
        for k in range(4):
            ob[row, pl.ds(k * 16, 16)] = cv[pl.ds(CLS0 + 16 * k, 16)]

    def mkidx(gi, _):
        rbase = gi * 16
        for t in range(3):
            iv = icv[t, pl.ds(rbase, 16)]
            idsall[pl.ds(gi * 144 + t * 16, 16)] = iv + OFFC[t]
        for t in range(6):
            iv = ipv[t, pl.ds(rbase, 16)]
            idsall[pl.ds(gi * 144 + 48 + t * 16, 16)] = iv + (PAY0 + t * 4)
        return 0

    lax.fori_loop(0, GRP, mkidx, 0)

    def fetch(gi, p):
        pltpu.async_copy(tab_hbm.at[idsall.at[pl.ds(gi * 144, 48)]],
                         gbuf.at[pl.ds(p * 144, 48), :], sem_in.at[p])
        pltpu.async_copy(tab_hbm.at[idsall.at[pl.ds(gi * 144 + 48, 96)]],
                         gbuf.at[pl.ds(p * 144 + 48, 96), :], sem_in.at[p])

    def fwait(p):
        pltpu.make_async_copy(tab_hbm.at[pl.ds(0, 48), :],
                              gbuf.at[pl.ds(0, 48), :], sem_in.at[p]).wait()
        pltpu.make_async_copy(tab_hbm.at[pl.ds(0, 96), :],
                              gbuf.at[pl.ds(0, 96), :], sem_in.at[p]).wait()

    def sfetch(gi, p):
        pltpu.async_copy(srep_hbm.at[:, pl.ds((base + gi * 16) * 16, 256)],
                         sbuf.at[pl.ds(p * 24, 24), :], sem_s.at[p])

    def swait(p):
        pltpu.make_async_copy(srep_hbm.at[:, pl.ds(0, 256)],
                              sbuf.at[pl.ds(0, 24), :], sem_s.at[p]).wait()

    c2p = cv[pl.ds(C2PSPL, 16)]
    c2n = cv[pl.ds(C2NSPL, 16)]
    wp = [cv[pl.ds(WPAY + 16 * k, 16)] for k in range(4)]
    bl = [cv[pl.ds(BLN + 16 * k, 16)] for k in range(4)]
    wn = [cv[pl.ds(WNUM + 16 * k, 16)] for k in range(4)]

    fetch(0, 0)
    fetch(1, 1)
    sfetch(0, 0)

    def group(gi, _):
        p = gi % 2
        rp = gi % 3
        brow0 = p * 16

        fwait(rp)
        swait(p)

        @pl.when(gi + 2 < GRP)
        def _prefetch_next():
            fetch(gi + 2, (gi + 2) % 3)

        @pl.when(gi + 1 < GRP)
        def _prefetch_s():
            sfetch(gi + 1, 1 - p)

        @pl.when(gi >= 2)
        def _wait_prev():
            pltpu.make_async_copy(
                ob.at[pl.ds(brow0, 16), :],
                out_hbm.at[pl.ds(0, 16), :],
                sems.at[p]).wait()

        go = rp * 144
        so = p * 24

        for t in range(3):
            def catj(j, _, t=t):
                gr = go + t * 16 + j
                brow = brow0 + j
                for k in range(4):
                    ob[brow, pl.ds((1 + t) * 64 + 16 * k, 16)] = \
                        gbuf[gr, pl.ds(16 * k, 16)]
                return 0
            lax.fori_loop(0, 16, catj, 0, unroll=4)

        for t in range(6):
            def payj(j, _, t=t):
                gr = go + 48 + t * 16 + j
                s = sbuf[so + t, pl.ds(j * 16, 16)]
                c0 = gbuf[gr, pl.ds(64, 16)]
                c1 = gbuf[gr, pl.ds(80, 16)]
                r = _rsqrt16((c2p * s + c1) * s + c0)
                brow = brow0 + j
                for k in range(4):
                    a = gbuf[gr, pl.ds(16 * k, 16)]
                    ob[brow, pl.ds((4 + t) * 64 + 16 * k, 16)] = \
                        (a + s * wp[k]) * r + bl[k]
                return 0
            lax.fori_loop(0, 16, payj, 0, unroll=4)

        for t in range(14):
            ak = [cv[pl.ds(ANUM + t * 64 + 16 * k, 16)] for k in range(4)]
            c0 = cv[pl.ds(C0NSPL + t * 16, 16)]
            c1 = cv[pl.ds(C1NSPL + t * 16, 16)]

            def numj(j, _, t=t, ak=ak, c0=c0, c1=c1):
                s = sbuf[so + 6 + t, pl.ds(j * 16, 16)]
                r = _rsqrt16((c2n * s + c1) * s + c0)
                brow = brow0 + j
                for k in range(4):
                    ob[brow, pl.ds((10 + t) * 64 + 16 * k, 16)] = \
                        (ak[k] + s * wn[k]) * r + bl[k]
                return 0
            lax.fori_loop(0, 16, numj, 0, unroll=4)

        pltpu.async_copy(
            ob.at[pl.ds(brow0, 16), :],
            out_hbm.at[pl.ds(base + gi * 16, 16), :],
            sems.at[p])
        return 0

    lax.fori_loop(0, GRP, group, 0)
    pltpu.make_async_copy(ob.at[pl.ds(0, 16), :],
                          out_hbm.at[pl.ds(0, 16), :], sems.at[0]).wait()
    pltpu.make_async_copy(ob.at[pl.ds(16, 16), :],
                          out_hbm.at[pl.ds(0, 16), :], sems.at[1]).wait()


@jax.jit
def _run_sc(tab, srep, ic, ip, consts):
    mesh = plsc.VectorSubcoreMesh(core_axis_name="c", subcore_axis_name="s",
                                  num_cores=2, num_subcores=16)
    k = pl.kernel(
        _sc_body,
        out_type=jax.ShapeDtypeStruct((B, ROWW), jnp.float32),
        mesh=mesh,
        compiler_params=pltpu.CompilerParams(needs_layout_passes=False),
        scratch_types=[
            pltpu.VMEM((3, RPW), jnp.int32),
            pltpu.VMEM((6, RPW), jnp.int32),
            pltpu.VMEM((NCONST,), jnp.float32),
            pltpu.VMEM((32, ROWW), jnp.float32),
            pltpu.VMEM((432, TROW), jnp.float32),
            pltpu.VMEM((48, 256), jnp.float32),
            pltpu.VMEM((GRP * 144,), jnp.int32),
            pltpu.SemaphoreType.DMA((2,)),
            pltpu.SemaphoreType.DMA((3,)),
            pltpu.SemaphoreType.DMA((2,)),
        ],
    )
    return k(tab, srep, ic, ip, consts)


def kernel(cat_idx_sex, cat_idx_education, cat_idx_marriage, pay_state_ids,
           pay_severities, num_values, emb_sex, emb_education, emb_marriage,
           pay_state_table, sev_W, sev_b, num_feat_table, val_W, val_b,
           pos_table, cls_token, ln_g, ln_b):
    f32 = jnp.float32
    g = ln_g.astype(f32)
    bln = ln_b.astype(f32)
    pos = pos_table.astype(f32)
    eps = 1e-5

    rows = jnp.concatenate([
        (cls_token[0, 0] + pos[0])[None],
        emb_sex + pos[1], emb_education + pos[2], emb_marriage + pos[3],
    ], axis=0)
    mu = rows.mean(-1, keepdims=True)
    var = ((rows - mu) ** 2).mean(-1, keepdims=True)
    lncat = (rows - mu) * lax.rsqrt(var + eps) * g + bln

    w_pay = sev_W[:, 0]
    a_pay = pay_state_table[None, :, :] + sev_b + pos[4:10][:, None, :]
    ah_pay = a_pay - a_pay.mean(-1, keepdims=True)
    wh_pay = w_pay - w_pay.mean()
    c0_pay = (ah_pay ** 2).mean(-1) + eps
    c1_pay = 2.0 * (ah_pay * wh_pay).mean(-1)
    c2_pay = (wh_pay ** 2).mean()

    w_num = val_W[:, 0]
    a_num = num_feat_table + val_b + pos[10:24]
    ah_num = a_num - a_num.mean(-1, keepdims=True)
    wh_num = w_num - w_num.mean()
    c0_num = (ah_num ** 2).mean(-1) + eps
    c1_num = 2.0 * (ah_num * wh_num).mean(-1)
    c2_num = (wh_num ** 2).mean()

    cat_rows = jnp.concatenate(
        [lncat, jnp.zeros((14, 64), f32)], axis=1)
    pay_rows = jnp.concatenate([
        (ah_pay * g).reshape(24, D),
        jnp.repeat(c0_pay.reshape(24, 1), 16, axis=1),
        jnp.repeat(c1_pay.reshape(24, 1), 16, axis=1),
        jnp.zeros((24, 32), f32),
    ], axis=1)
    tab = jnp.concatenate([cat_rows, pay_rows], axis=0)

    consts = jnp.concatenate([
        lncat[0],
        wh_pay * g, wh_num * g, bln,
        (ah_num * g).reshape(-1),
        jnp.repeat(c0_num, 16), jnp.repeat(c1_num, 16),
        jnp.full((16,), c2_pay, f32), jnp.full((16,), c2_num, f32),
    ])

    s_all = jnp.concatenate([pay_severities.T, num_values.T], axis=0)
    srep = jnp.concatenate([jnp.repeat(s_all.astype(f32), 16, axis=1),
                            jnp.zeros((4, 16 * B), f32)])
    ic = jnp.stack([cat_idx_sex, cat_idx_education,
                    cat_idx_marriage]).astype(jnp.int32)
    ip = pay_state_ids.T.astype(jnp.int32)

    out = _run_sc(tab, srep, ic, ip, consts)
    return out.reshape(B, 24, D)


# speedup vs baseline: 1.6798x; 1.6798x over previous
import functools

import jax
import jax.numpy as jnp
from jax import lax
from jax.experimental import pallas as pl
from jax.experimental.pallas import tpu as pltpu
from jax.experimental.pallas import tpu_sc as plsc

D = 64
B = 16384
NW = 32
RPW = B // NW
GRP = RPW // 16
ROWW = 24 * D
BUFW = 16 * ROWW

LNCAT = 0
APAY = 896
WPAY = APAY + 1536
WNUM = WPAY + 64
BLN = WNUM + 64
ANUM = BLN + 64
C0PAY = ANUM + 896
C1PAY = C0PAY + 32
C0NSPL = C1PAY + 32
C1NSPL = C0NSPL + 224
C2PSPL = C1NSPL + 224
C2NSPL = C2PSPL + 16
NCONST = C2NSPL + 16
OFFC = (1, 3, 10)
RQP = 10240
IRP = 2304


def _rsqrt16(x):
    i = plsc.bitcast(x, jnp.int32)
    i = jnp.int32(0x5F3759DF) - (i >> 1)
    y = plsc.bitcast(i, jnp.float32)
    xh = x * 0.5
    for _ in range(3):
        y = y * (1.5 - xh * y * y)
    return y


def _sc_body(s_hbm, ic_hbm, ip_hbm, c_hbm, out_hbm,
             sv, icv, ipv, cv, ob, rrep, irep, sems):
    wid = lax.axis_index("s") * 2 + lax.axis_index("c")
    base = wid * RPW
    pltpu.sync_copy(c_hbm, cv)
    pltpu.sync_copy(s_hbm.at[:, pl.ds(base, RPW)], sv)
    pltpu.sync_copy(ic_hbm.at[:, pl.ds(base, RPW)], icv)
    pltpu.sync_copy(ip_hbm.at[:, pl.ds(base, RPW)], ipv)

    for row in range(32):
        for k in range(4):
            ob[row, pl.ds(k * 16, 16)] = cv[pl.ds(LNCAT + k * 16, 16)]

    iota = lax.iota(jnp.int32, 16)

    def repl_f(vec, off):
        def body(c, _):
            dg = iota * 16 + ((iota + c) & 15)
            plsc.store_scatter(rrep, [dg + off], vec)
            return 0
        lax.fori_loop(0, 16, body, 0, unroll=4)

    def repl_i(vec, off):
        def body(c, _):
            dg = iota * 16 + ((iota + c) & 15)
            plsc.store_scatter(irep, [dg + off], vec)
            return 0
        lax.fori_loop(0, 16, body, 0, unroll=4)

    c2p = cv[pl.ds(C2PSPL, 16)]
    c2n = cv[pl.ds(C2NSPL, 16)]

    def phase1(gi):
        p = gi % 2
        rbase = gi * 16
        ro = p * RQP
        io = p * IRP
        for t in range(3):
            iv = icv[t, pl.ds(rbase, 16)]
            repl_i(iv * 64 + (LNCAT + OFFC[t] * 64), io + t * 256)
        for t in range(6):
            s = sv[t, pl.ds(rbase, 16)]
            iv = ipv[t, pl.ds(rbase, 16)]
            ci = iv + t * 4
            c0 = plsc.load_gather(cv, [ci + C0PAY])
            c1 = plsc.load_gather(cv, [ci + C1PAY])
            r = _rsqrt16((c2p * s + c1) * s + c0)
            repl_i(iv * 64 + (APAY + t * 256), io + (3 + t) * 256)
            repl_f(r, ro + t * 256)
            repl_f(s * r, ro + 5120 + t * 256)
        for t in range(14):
            s = sv[6 + t, pl.ds(rbase, 16)]
            c0 = cv[pl.ds(C0NSPL + t * 16, 16)]
            c1 = cv[pl.ds(C1NSPL + t * 16, 16)]
            r = _rsqrt16((c2n * s + c1) * s + c0)
            repl_f(r, ro + (6 + t) * 256)
            repl_f(s * r, ro + 5120 + (6 + t) * 256)

    wp = [cv[pl.ds(WPAY + 16 * k, 16)] for k in range(4)]
    bl = [cv[pl.ds(BLN + 16 * k, 16)] for k in range(4)]
    wn = [cv[pl.ds(WNUM + 16 * k, 16)] for k in range(4)]

    phase1(0)

    def group(gi, _):
        buf = gi % 2
        brow0 = buf * 16
        p = gi % 2
        ro = p * RQP
        io = p * IRP

        @pl.when(gi >= 2)
        def _wait_prev():
            pltpu.make_async_copy(
                ob.at[pl.ds(brow0, 16), :],
                out_hbm.at[pl.ds(0, 16), :],
                sems.at[buf]).wait()

        rbase = gi * 16

        for t in range(3):
            def catj(j, _, t=t):
                gbj = irep[pl.ds(io + t * 256 + j * 16, 16)]
                brow = brow0 + j
                for k in range(4):
                    v = plsc.load_gather(cv, [gbj + (iota + 16 * k)])
                    ob[brow, pl.ds((1 + t) * 64 + 16 * k, 16)] = v
                return 0
            lax.fori_loop(0, 16, catj, 0, unroll=4)

        for t in range(6):
            def payj(j, _, t=t):
                rj = rrep[pl.ds(ro + t * 256 + j * 16, 16)]
                qj = rrep[pl.ds(ro + 5120 + t * 256 + j * 16, 16)]
                abj = irep[pl.ds(io + (3 + t) * 256 + j * 16, 16)]
                brow = brow0 + j
                for k in range(4):
                    a = plsc.load_gather(cv, [abj + (iota + 16 * k)])
                    ob[brow, pl.ds((4 + t) * 64 + 16 * k, 16)] = \
                        a * rj + wp[k] * qj + bl[k]
                return 0
            lax.fori_loop(0, 16, payj, 0, unroll=4)

        for t in range(14):
            ak = [cv[pl.ds(ANUM + t * 64 + 16 * k, 16)] for k in range(4)]

            def numj(j, _, t=t, ak=ak):
                rj = rrep[pl.ds(ro + (6 + t) * 256 + j * 16, 16)]
                qj = rrep[pl.ds(ro + 5120 + (6 + t) * 256 + j * 16, 16)]
                brow = brow0 + j
                for k in range(4):
                    ob[brow, pl.ds((10 + t) * 64 + 16 * k, 16)] = \
                        ak[k] * rj + wn[k] * qj + bl[k]
                return 0
            lax.fori_loop(0, 16, numj, 0, unroll=4)

        pltpu.async_copy(
            ob.at[pl.ds(brow0, 16), :],
            out_hbm.at[pl.ds(base + rbase, 16), :],
            sems.at[buf])

        @pl.when(gi + 1 < GRP)
        def _next_phase1():
            phase1(gi + 1)

        return 0

    lax.fori_loop(0, GRP, group, 0)
    pltpu.make_async_copy(ob.at[pl.ds(0, 16), :],
                          out_hbm.at[pl.ds(0, 16), :], sems.at[0]).wait()
    pltpu.make_async_copy(ob.at[pl.ds(16, 16), :],
                          out_hbm.at[pl.ds(0, 16), :], sems.at[1]).wait()


@functools.partial(jax.jit, static_argnums=())
def _run_sc(s_all, ic, ip, consts):
    mesh = plsc.VectorSubcoreMesh(core_axis_name="c", subcore_axis_name="s",
                                  num_cores=2, num_subcores=16)
    k = pl.kernel(
        _sc_body,
        out_type=jax.ShapeDtypeStruct((B, ROWW), jnp.float32),
        mesh=mesh,
        compiler_params=pltpu.CompilerParams(needs_layout_passes=False),
        scratch_types=[
            pltpu.VMEM((20, RPW), jnp.float32),
            pltpu.VMEM((3, RPW), jnp.int32),
            pltpu.VMEM((6, RPW), jnp.int32),
            pltpu.VMEM((NCONST,), jnp.float32),
            pltpu.VMEM((32, ROWW), jnp.float32),
            pltpu.VMEM((2 * RQP,), jnp.float32),
            pltpu.VMEM((2 * IRP,), jnp.int32),
            pltpu.SemaphoreType.DMA((2,)),
        ],
    )
    return k(s_all, ic, ip, consts)


def kernel(cat_idx_sex, cat_idx_education, cat_idx_marriage, pay_state_ids,
           pay_severities, num_values, emb_sex, emb_education, emb_marriage,
           pay_state_table, sev_W, sev_b, num_feat_table, val_W, val_b,
           pos_table, cls_token, ln_g, ln_b):
    f32 = jnp.float32
    g = ln_g.astype(f32)
    bln = ln_b.astype(f32)
    pos = pos_table.astype(f32)
    eps = 1e-5

    rows = jnp.concatenate([
        (cls_token[0, 0] + pos[0])[None],
        emb_sex + pos[1], emb_education + pos[2], emb_marriage + pos[3],
    ], axis=0)
    mu = rows.mean(-1, keepdims=True)
    var = ((rows - mu) ** 2).mean(-1, keepdims=True)
    lncat = (rows - mu) * lax.rsqrt(var + eps) * g + bln

    w_pay = sev_W[:, 0]
    a_pay = pay_state_table[None, :, :] + sev_b + pos[4:10][:, None, :]
    ah_pay = a_pay - a_pay.mean(-1, keepdims=True)
    wh_pay = w_pay - w_pay.mean()
    c0_pay = (ah_pay ** 2).mean(-1) + eps
    c1_pay = 2.0 * (ah_pay * wh_pay).mean(-1)
    c2_pay = (wh_pay ** 2).mean()

    w_num = val_W[:, 0]
    a_num = num_feat_table + val_b + pos[10:24]
    ah_num = a_num - a_num.mean(-1, keepdims=True)
    wh_num = w_num - w_num.mean()
    c0_num = (ah_num ** 2).mean(-1) + eps
    c1_num = 2.0 * (ah_num * wh_num).mean(-1)
    c2_num = (wh_num ** 2).mean()

    pad8 = jnp.zeros((8,), f32)
    consts = jnp.concatenate([
        lncat.reshape(-1),
        (ah_pay * g).reshape(-1),
        wh_pay * g, wh_num * g, bln,
        (ah_num * g).reshape(-1),
        c0_pay.reshape(-1), pad8, c1_pay.reshape(-1), pad8,
        jnp.repeat(c0_num, 16), jnp.repeat(c1_num, 16),
        jnp.full((16,), c2_pay, f32), jnp.full((16,), c2_num, f32),
    ])

    s_all = jnp.concatenate([pay_severities.T, num_values.T], axis=0)
    ic = jnp.stack([cat_idx_sex, cat_idx_education,
                    cat_idx_marriage]).astype(jnp.int32)
    ip = pay_state_ids.T.astype(jnp.int32)

    out = _run_sc(s_all.astype(f32), ic, ip, consts)
    return out.reshape(B, 24, D)


# speedup vs baseline: 1.7317x; 1.0309x over previous
import functools

import jax
import jax.numpy as jnp
from jax import lax
from jax.experimental import pallas as pl
from jax.experimental.pallas import tpu as pltpu
from jax.experimental.pallas import tpu_sc as plsc

D = 64
B = 16384
NW = 32
RPW = B // NW
GRP = RPW // 16
ROWW = 24 * D
BUFW = 16 * ROWW

LNCAT = 0
APAY = 896
WPAY = APAY + 1536
WNUM = WPAY + 64
BLN = WNUM + 64
ANUM = BLN + 64
C0PAY = ANUM + 896
C1PAY = C0PAY + 32
C0NSPL = C1PAY + 32
C1NSPL = C0NSPL + 224
C2PSPL = C1NSPL + 224
C2NSPL = C2PSPL + 16
NCONST = C2NSPL + 16
OFFC = (1, 3, 10)
RQP = 10240
IRP = 2304


def _rsqrt16(x):
    i = plsc.bitcast(x, jnp.int32)
    i = jnp.int32(0x5F3759DF) - (i >> 1)
    y = plsc.bitcast(i, jnp.float32)
    return y * (1.5 - (x * 0.5) * y * y)


def _sc_body(s_hbm, ic_hbm, ip_hbm, c_hbm, out_hbm,
             sv, icv, ipv, cv, ob, rrep, irep, sems):
    wid = lax.axis_index("s") * 2 + lax.axis_index("c")
    base = wid * RPW
    pltpu.sync_copy(c_hbm, cv)
    pltpu.sync_copy(s_hbm.at[:, pl.ds(base, RPW)], sv)
    pltpu.sync_copy(ic_hbm.at[:, pl.ds(base, RPW)], icv)
    pltpu.sync_copy(ip_hbm.at[:, pl.ds(base, RPW)], ipv)

    for row in range(32):
        for k in range(4):
            ob[row, pl.ds(k * 16, 16)] = cv[pl.ds(LNCAT + k * 16, 16)]

    iota = lax.iota(jnp.int32, 16)

    def repl_f(vec, off):
        def body(c, _):
            dg = iota * 16 + ((iota + c) & 15)
            plsc.store_scatter(rrep, [dg + off], vec)
            return 0
        lax.fori_loop(0, 16, body, 0, unroll=4)

    def repl_i(vec, off):
        def body(c, _):
            dg = iota * 16 + ((iota + c) & 15)
            plsc.store_scatter(irep, [dg + off], vec)
            return 0
        lax.fori_loop(0, 16, body, 0, unroll=4)

    c2p = cv[pl.ds(C2PSPL, 16)]
    c2n = cv[pl.ds(C2NSPL, 16)]

    def phase1(gi):
        p = gi % 2
        rbase = gi * 16
        ro = p * RQP
        io = p * IRP
        for t in range(3):
            iv = icv[t, pl.ds(rbase, 16)]
            repl_i(iv * 64 + (LNCAT + OFFC[t] * 64), io + t * 256)
        for t in range(6):
            s = sv[t, pl.ds(rbase, 16)]
            iv = ipv[t, pl.ds(rbase, 16)]
            ci = iv + t * 4
            c0 = plsc.load_gather(cv, [ci + C0PAY])
            c1 = plsc.load_gather(cv, [ci + C1PAY])
            r = _rsqrt16((c2p * s + c1) * s + c0)
            repl_i(iv * 64 + (APAY + t * 256), io + (3 + t) * 256)
            repl_f(r, ro + t * 256)
            repl_f(s * r, ro + 5120 + t * 256)
        for t in range(14):
            s = sv[6 + t, pl.ds(rbase, 16)]
            c0 = cv[pl.ds(C0NSPL + t * 16, 16)]
            c1 = cv[pl.ds(C1NSPL + t * 16, 16)]
            r = _rsqrt16((c2n * s + c1) * s + c0)
            repl_f(r, ro + (6 + t) * 256)
            repl_f(s * r, ro + 5120 + (6 + t) * 256)

    wp = [cv[pl.ds(WPAY + 16 * k, 16)] for k in range(4)]
    bl = [cv[pl.ds(BLN + 16 * k, 16)] for k in range(4)]
    wn = [cv[pl.ds(WNUM + 16 * k, 16)] for k in range(4)]

    phase1(0)

    def group(gi, _):
        buf = gi % 2
        brow0 = buf * 16
        p = gi % 2
        ro = p * RQP
        io = p * IRP

        @pl.when(gi >= 2)
        def _wait_prev():
            pltpu.make_async_copy(
                ob.at[pl.ds(brow0, 16), :],
                out_hbm.at[pl.ds(0, 16), :],
                sems.at[buf]).wait()

        rbase = gi * 16

        for t in range(3):
            def catj(j, _, t=t):
                gbj = irep[pl.ds(io + t * 256 + j * 16, 16)]
                brow = brow0 + j
                for k in range(4):
                    v = plsc.load_gather(cv, [gbj + (iota + 16 * k)])
                    ob[brow, pl.ds((1 + t) * 64 + 16 * k, 16)] = v
                return 0
            lax.fori_loop(0, 16, catj, 0, unroll=8)

        for t in range(6):
            def payj(j, _, t=t):
                rj = rrep[pl.ds(ro + t * 256 + j * 16, 16)]
                qj = rrep[pl.ds(ro + 5120 + t * 256 + j * 16, 16)]
                abj = irep[pl.ds(io + (3 + t) * 256 + j * 16, 16)]
                brow = brow0 + j
                for k in range(4):
                    a = plsc.load_gather(cv, [abj + (iota + 16 * k)])
                    ob[brow, pl.ds((4 + t) * 64 + 16 * k, 16)] = \
                        a * rj + wp[k] * qj + bl[k]
                return 0
            lax.fori_loop(0, 16, payj, 0, unroll=8)

        for t in range(14):
            ak = [cv[pl.ds(ANUM + t * 64 + 16 * k, 16)] for k in range(4)]

            def numj(j, _, t=t, ak=ak):
                rj = rrep[pl.ds(ro + (6 + t) * 256 + j * 16, 16)]
                qj = rrep[pl.ds(ro + 5120 + (6 + t) * 256 + j * 16, 16)]
                brow = brow0 + j
                for k in range(4):
                    ob[brow, pl.ds((10 + t) * 64 + 16 * k, 16)] = \
                        ak[k] * rj + wn[k] * qj + bl[k]
                return 0
            lax.fori_loop(0, 16, numj, 0, unroll=8)

        pltpu.async_copy(
            ob.at[pl.ds(brow0, 16), :],
            out_hbm.at[pl.ds(base + rbase, 16), :],
            sems.at[buf])

        @pl.when(gi + 1 < GRP)
        def _next_phase1():
            phase1(gi + 1)

        return 0

    lax.fori_loop(0, GRP, group, 0)
    pltpu.make_async_copy(ob.at[pl.ds(0, 16), :],
                          out_hbm.at[pl.ds(0, 16), :], sems.at[0]).wait()
    pltpu.make_async_copy(ob.at[pl.ds(16, 16), :],
                          out_hbm.at[pl.ds(0, 16), :], sems.at[1]).wait()


@functools.partial(jax.jit, static_argnums=())
def _run_sc(s_all, ic, ip, consts):
    mesh = plsc.VectorSubcoreMesh(core_axis_name="c", subcore_axis_name="s",
                                  num_cores=2, num_subcores=16)
    k = pl.kernel(
        _sc_body,
        out_type=jax.ShapeDtypeStruct((B, ROWW), jnp.float32),
        mesh=mesh,
        compiler_params=pltpu.CompilerParams(needs_layout_passes=False),
        scratch_types=[
            pltpu.VMEM((20, RPW), jnp.float32),
            pltpu.VMEM((3, RPW), jnp.int32),
            pltpu.VMEM((6, RPW), jnp.int32),
            pltpu.VMEM((NCONST,), jnp.float32),
            pltpu.VMEM((32, ROWW), jnp.float32),
            pltpu.VMEM((2 * RQP,), jnp.float32),
            pltpu.VMEM((2 * IRP,), jnp.int32),
            pltpu.SemaphoreType.DMA((2,)),
        ],
    )
    return k(s_all, ic, ip, consts)


def kernel(cat_idx_sex, cat_idx_education, cat_idx_marriage, pay_state_ids,
           pay_severities, num_values, emb_sex, emb_education, emb_marriage,
           pay_state_table, sev_W, sev_b, num_feat_table, val_W, val_b,
           pos_table, cls_token, ln_g, ln_b):
    f32 = jnp.float32
    g = ln_g.astype(f32)
    bln = ln_b.astype(f32)
    pos = pos_table.astype(f32)
    eps = 1e-5

    rows = jnp.concatenate([
        (cls_token[0, 0] + pos[0])[None],
        emb_sex + pos[1], emb_education + pos[2], emb_marriage + pos[3],
    ], axis=0)
    mu = rows.mean(-1, keepdims=True)
    var = ((rows - mu) ** 2).mean(-1, keepdims=True)
    lncat = (rows - mu) * lax.rsqrt(var + eps) * g + bln

    w_pay = sev_W[:, 0]
    a_pay = pay_state_table[None, :, :] + sev_b + pos[4:10][:, None, :]
    ah_pay = a_pay - a_pay.mean(-1, keepdims=True)
    wh_pay = w_pay - w_pay.mean()
    c0_pay = (ah_pay ** 2).mean(-1) + eps
    c1_pay = 2.0 * (ah_pay * wh_pay).mean(-1)
    c2_pay = (wh_pay ** 2).mean()

    w_num = val_W[:, 0]
    a_num = num_feat_table + val_b + pos[10:24]
    ah_num = a_num - a_num.mean(-1, keepdims=True)
    wh_num = w_num - w_num.mean()
    c0_num = (ah_num ** 2).mean(-1) + eps
    c1_num = 2.0 * (ah_num * wh_num).mean(-1)
    c2_num = (wh_num ** 2).mean()

    pad8 = jnp.zeros((8,), f32)
    consts = jnp.concatenate([
        lncat.reshape(-1),
        (ah_pay * g).reshape(-1),
        wh_pay * g, wh_num * g, bln,
        (ah_num * g).reshape(-1),
        c0_pay.reshape(-1), pad8, c1_pay.reshape(-1), pad8,
        jnp.repeat(c0_num, 16), jnp.repeat(c1_num, 16),
        jnp.full((16,), c2_pay, f32), jnp.full((16,), c2_num, f32),
    ])

    s_all = jnp.concatenate([pay_severities.T, num_values.T], axis=0)
    ic = jnp.stack([cat_idx_sex, cat_idx_education,
                    cat_idx_marriage]).astype(jnp.int32)
    ip = pay_state_ids.T.astype(jnp.int32)

    out = _run_sc(s_all.astype(f32), ic, ip, consts)
    return out.reshape(B, 24, D)
